# Initial kernel scaffold; baseline (speedup 1.0000x reference)
import jax
import jax.numpy as jnp
from jax.experimental import pallas as pl


def kernel(x_ego, x_nei, params):
    raise NotImplementedError("write your pallas kernel here")


# speedup vs baseline: 1.8829x; 1.8829x over previous
import functools

import numpy as np
import jax
import jax.numpy as jnp
from jax.experimental import pallas as pl

T_H, T_F, D, D_HALF, D_NOISE, INSIGHTS, CAPACITY, TRAJ = 8, 12, 128, 64, 16, 8, 8, 2
H, DFF, N_LAYERS = 2, 128, 2
B, N_NEI = 1024, 64
SB = 8
PAIRS = SB * CAPACITY
NSEQ = 2 * PAIRS
NTOK = NSEQ * T_H
NTOKH = PAIRS * T_H
ROWS = SB * N_NEI
FC = T_F * TRAJ
OUTW = INSIGHTS * FC

_HI = jax.lax.Precision.HIGHEST


def _pos_encoding():
    pos = np.arange(T_H)[:, None].astype(np.float32)
    i = np.arange(D)[None, :].astype(np.float32)
    angle = pos / np.power(10000.0, (2.0 * np.floor(i / 2.0)) / D)
    pe = np.zeros((T_H, D), dtype=np.float32)
    pe[:, 0::2] = np.sin(angle[:, 0::2])
    pe[:, 1::2] = np.cos(angle[:, 1::2])
    return pe


def _fit_mats():
    t = np.arange(T_H, dtype=np.float64)
    tm = t.mean()
    denom = ((t - tm) ** 2).sum()
    tf = np.arange(T_H, T_H + T_F, dtype=np.float64)
    Fy = (1.0 / T_H) + np.outer(t - tm, tf - tm) / denom
    C = np.eye(T_H)
    C[T_H - 1, :] -= 1.0
    Myl1 = C @ Fy
    def expand(M1, ncols):
        M = np.zeros((T_H * TRAJ, ncols * TRAJ), dtype=np.float64)
        for c in range(TRAJ):
            M[c::TRAJ, c::TRAJ] = M1
        return M
    Msm = np.zeros((T_H * TRAJ, 4), dtype=np.float64)
    for c in range(TRAJ):
        Msm[c::TRAJ, c] = (t - tm) / denom
        Msm[c::TRAJ, 2 + c] = 1.0 / T_H
    return (Msm.astype(np.float32),
            expand(Myl1, T_F).astype(np.float32),
            expand(Fy, T_F).astype(np.float32))


def _consts():
    f32 = np.float32
    ar = np.arange
    lt = (ar(N_NEI)[:, None] < ar(N_NEI)[None, :]).astype(f32)
    osc = (ar(PAIRS)[:, None] // CAPACITY == ar(SB)[None, :]).astype(f32)
    mg = (ar(PAIRS)[:, None] // CAPACITY == ar(ROWS)[None, :] // N_NEI).astype(f32)
    kp = (ar(PAIRS)[:, None] % CAPACITY).astype(f32)
    e8s = (ar(NTOK)[:, None] // T_H == ar(NSEQ)[None, :]).astype(f32)
    e8p = (ar(NTOKH)[:, None] // T_H == ar(PAIRS)[None, :]).astype(f32)
    lx = (ar(T_H * TRAJ)[None, :] == TRAJ * (ar(NTOKH)[:, None] % T_H)).astype(f32)
    ly = (ar(T_H * TRAJ)[None, :] == TRAJ * (ar(NTOKH)[:, None] % T_H) + 1).astype(f32)
    ttok = (ar(NTOK)[:, None] % T_H).astype(f32)
    e12 = (ar(FC)[None, :] // TRAJ == ar(T_F)[:, None]).astype(f32)
    e2 = (ar(FC)[None, :] % TRAJ == ar(TRAJ)[:, None]).astype(f32)
    exp8 = (ar(OUTW)[None, :] // FC == ar(INSIGHTS)[:, None]).astype(f32)
    tile24 = (ar(OUTW)[None, :] % FC == ar(FC)[:, None]).astype(f32)
    tile2 = (ar(OUTW)[None, :] % TRAJ == ar(TRAJ)[:, None]).astype(f32)
    erow = (ar(ROWS)[:, None] // N_NEI == ar(SB)[None, :]).astype(f32)
    lpick = (ar(N_NEI)[None, :] == ar(ROWS)[:, None] % N_NEI).astype(f32)
    msc = (ar(ROWS)[:, None] // N_NEI == ar(PAIRS)[None, :] // CAPACITY).astype(f32)
    kpat = (ar(PAIRS)[None, :] % CAPACITY).astype(f32)
    return dict(lt=lt, osc=osc, mg=mg, kp=kp, e8s=e8s, e8p=e8p, lx=lx,
                ly=ly, ttok=ttok, e12=e12, e2=e2, exp8=exp8,
                tile24=tile24, tile2=tile2, erow=erow, lpick=lpick,
                msc=msc, kpat=kpat)


_CONST_NAMES = ('lt', 'osc', 'mg', 'kp', 'e8s', 'e8p', 'lx', 'ly', 'ttok',
                'e12', 'e2', 'exp8', 'tile24', 'tile2', 'erow', 'lpick',
                'msc', 'kpat')


def _ln(x, lp):
    m = x.mean(-1, keepdims=True)
    d = x - m
    v = (d * d).mean(-1, keepdims=True)
    return lp['g'] * d / jnp.sqrt(v + 1e-6) + lp['b']


def _mha(q_in, kv_in, p):
    dh = D // H
    q = jnp.dot(q_in, p['wq'], preferred_element_type=jnp.float32) + p['bq']
    k = jnp.dot(kv_in, p['wk'], preferred_element_type=jnp.float32) + p['bk']
    v = jnp.dot(kv_in, p['wv'], preferred_element_type=jnp.float32) + p['bv']
    n = q_in.shape[0] // T_H
    outs = []
    for h in range(H):
        sl = slice(h * dh, (h + 1) * dh)
        qh = q[:, sl].reshape(n, T_H, dh)
        kh = k[:, sl].reshape(n, T_H, dh)
        vh = v[:, sl].reshape(n, T_H, dh)
        sc = jnp.einsum('nsd,ntd->nst', qh, kh,
                        preferred_element_type=jnp.float32) / jnp.sqrt(jnp.float32(dh))
        mx = jnp.max(sc, axis=-1, keepdims=True)
        e = jnp.exp(sc - mx)
        a = e / jnp.sum(e, axis=-1, keepdims=True)
        oh = jnp.einsum('nst,ntd->nsd', a, vh,
                        preferred_element_type=jnp.float32)
        outs.append(oh.reshape(n * T_H, dh))
    o = jnp.concatenate(outs, axis=-1)
    return jnp.dot(o, p['wo'], preferred_element_type=jnp.float32) + p['bo']


def _ffn(x, p):
    h = jnp.maximum(jnp.dot(x, p['w1'], preferred_element_type=jnp.float32) + p['b1'], 0.0)
    return jnp.dot(h, p['w2'], preferred_element_type=jnp.float32) + p['b2']


def _block_body(treedef, *refs):
    (xe_ref, xn_ref, xnT_ref, xet_ref, ze_ref, zn_ref, pe_ref,
     msm_ref, myl_ref, mb_ref) = refs[:10]
    c = {name: refs[10 + i][...] for i, name in enumerate(_CONST_NAMES)}
    param_refs = refs[10 + len(_CONST_NAMES):-1]
    out_ref = refs[-1]
    p = jax.tree.unflatten(treedef, [r[...] for r in param_refs])

    xe = xe_ref[...]
    xn = xn_ref[...]
    xnT = xnT_ref[...].reshape(SB, T_H * TRAJ, N_NEI)
    pe = pe_ref[...]

    valid = jnp.sum(jnp.abs(xnT), axis=1) > 0.0
    nlx = xnT[:, (T_H - 1) * TRAJ, :]
    nly = xnT[:, (T_H - 1) * TRAJ + 1, :]
    dx = xe[:, 14:15] - nlx
    dy = xe[:, 15:16] - nly
    dist = jnp.sqrt(dx * dx + dy * dy)
    iota64 = jax.lax.broadcasted_iota(jnp.int32, (SB, N_NEI), 1)
    cap = jnp.zeros((SB, N_NEI), jnp.bool_)
    Dm = dist
    for _ in range(CAPACITY):
        m = jnp.min(Dm, axis=1, keepdims=True)
        cand = jnp.where(Dm == m, iota64, N_NEI)
        amin = jnp.min(cand, axis=1, keepdims=True)
        pick = iota64 == amin
        cap = cap | pick
        Dm = jnp.where(pick, jnp.inf, Dm)
    sel = jnp.where(cap & valid, 1.0, 0.0)
    rank = jnp.dot(sel, c['lt'], precision=_HI)

    sel_pair = jnp.dot(c['osc'], sel, precision=_HI)
    rank_pair = jnp.dot(c['osc'], rank, precision=_HI)
    Gsel = jnp.concatenate([sel_pair] * SB, axis=1) * c['mg']
    Grank = jnp.concatenate([rank_pair] * SB, axis=1)
    G = jnp.where((Gsel > 0.5) & (Grank == c['kp']), 1.0, 0.0)
    xsel = jnp.dot(G, xn, precision=_HI)

    ego_seq = jnp.broadcast_to(xe[:, None, :], (SB, CAPACITY, T_H * TRAJ)
                               ).reshape(PAIRS, T_H * TRAJ)
    x_all = jnp.concatenate([ego_seq, xsel], axis=0)

    smm = jnp.dot(x_all, msm_ref[...], precision=_HI)
    slp = smm[:, 0:2]
    ref_seq = x_all[:, 14:16]
    ic = (smm[:, 2:4] - ref_seq) - slp * 3.5
    seqdat = jnp.concatenate([slp, ic, ref_seq], axis=1)
    tokdat = jnp.dot(c['e8s'], seqdat, precision=_HI)

    xet = xet_ref[...].reshape(SB, T_H, TRAJ)
    xe_tok = jnp.broadcast_to(xet[:, None, :, :], (SB, CAPACITY, T_H, TRAJ)
                              ).reshape(NTOKH, TRAJ)
    xsel_exp = jnp.dot(c['e8p'], xsel, precision=_HI)
    xsx = jnp.sum(xsel_exp * c['lx'], axis=1, keepdims=True)
    xsy = jnp.sum(xsel_exp * c['ly'], axis=1, keepdims=True)
    xsel_tok = jnp.concatenate([xsx, xsy], axis=1)
    x_tok = jnp.concatenate([xe_tok, xsel_tok], axis=0)

    xdiff_tok = (x_tok - tokdat[:, 4:6]) - (tokdat[:, 0:2] * c['ttok'] + tokdat[:, 2:4])

    ylin_nei = jnp.dot(xsel, myl_ref[...], precision=_HI)
    ref_nei = xsel[:, 14:16]

    f_diff = jnp.tanh(jnp.dot(xdiff_tok, p['w_ld'],
                              preferred_element_type=jnp.float32) + p['b_ld'])
    z_tok = jnp.concatenate([ze_ref[...], zn_ref[...]], axis=0)
    f_z = jnp.tanh(jnp.dot(z_tok, p['w_noise'],
                           preferred_element_type=jnp.float32) + p['b_noise'])
    f_final = jnp.concatenate([f_diff, f_z], axis=-1)

    pe_tok = jnp.broadcast_to(pe[None], (NSEQ, T_H, D)).reshape(NTOK, D)

    enc = jnp.dot(f_final, p['w_ei'], preferred_element_type=jnp.float32) + p['b_ei'] + pe_tok
    for lp in p['enc']:
        enc = _ln(enc + _mha(enc, enc, lp['att']), lp['ln1'])
        enc = _ln(enc + _ffn(enc, lp['ffn']), lp['ln2'])
    dec = jnp.dot(xdiff_tok, p['w_di'], preferred_element_type=jnp.float32) + p['b_di'] + pe_tok
    for lp in p['dec']:
        dec = _ln(dec + _mha(dec, dec, lp['satt']), lp['ln1'])
        dec = _ln(dec + _mha(dec, enc, lp['catt']), lp['ln2'])
        dec = _ln(dec + _ffn(dec, lp['ffn']), lp['ln3'])

    f_ego = dec[:NTOKH]
    f_nei = dec[NTOKH:]

    I_t = _ffn(f_ego, p['k1'])
    R_t = _ffn(f_nei, p['k2'])
    g_t = jnp.dot(f_nei, p['w_dec'], precision=_HI)

    Bf = (jnp.dot(R_t, c['e12'], precision=_HI)
          * jnp.dot(g_t, c['e2'], precision=_HI))
    y192 = (jnp.dot(I_t, c['exp8'], precision=_HI)
            * jnp.dot(Bf, c['tile24'], precision=_HI))
    y_pairs = y192.reshape(PAIRS, T_H, OUTW).sum(axis=1)
    y_full = (y_pairs
              + jnp.dot(p['b_dec'], c['tile2'], precision=_HI)
              + jnp.dot(ylin_nei, c['tile24'], precision=_HI)
              + jnp.dot(ref_nei, c['tile2'], precision=_HI))

    y_base = jnp.dot(jnp.dot(xn, mb_ref[...], precision=_HI),
                     c['tile24'], precision=_HI)
    sel_exp = jnp.dot(c['erow'], sel, precision=_HI)
    rank_exp = jnp.dot(c['erow'], rank, precision=_HI)
    sel_row = jnp.sum(sel_exp * c['lpick'], axis=1, keepdims=True)
    rank_row = jnp.sum(rank_exp * c['lpick'], axis=1, keepdims=True)
    S = jnp.where((c['msc'] > 0.5) & (sel_row > 0.5)
                  & (rank_row == c['kpat']), 1.0, 0.0)
    y_scat = jnp.dot(S, y_full, precision=_HI)
    out_ref[...] = jnp.where(sel_row > 0.5, y_scat, y_base)


def kernel(x_ego, x_nei, params):
    leaves, treedef = jax.tree.flatten(params)
    leaves = [l.reshape(1, -1) if l.ndim == 1 else l for l in leaves]

    xe2 = x_ego.reshape(B, T_H * TRAJ)
    xn2 = x_nei.reshape(B * N_NEI, T_H * TRAJ)
    xnT = x_nei.transpose(0, 2, 3, 1).reshape(B * T_H * TRAJ, N_NEI)
    xet = x_ego.reshape(B * T_H, TRAJ)
    z = jax.random.normal(jax.random.key(7), (2 * B * CAPACITY, T_H, D_NOISE),
                          dtype=jnp.float32)
    ze = z[:B * CAPACITY].reshape(B * CAPACITY * T_H, D_NOISE)
    zn = z[B * CAPACITY:].reshape(B * CAPACITY * T_H, D_NOISE)
    pe = jnp.asarray(_pos_encoding())
    msm, myl, mb = (jnp.asarray(m) for m in _fit_mats())
    cmats = _consts()
    cvals = [jnp.asarray(cmats[n]) for n in _CONST_NAMES]

    nblk = B // SB
    fixed = lambda shape: pl.BlockSpec(shape, lambda b: (0,) * len(shape))
    in_specs = [
        pl.BlockSpec((SB, T_H * TRAJ), lambda b: (b, 0)),
        pl.BlockSpec((ROWS, T_H * TRAJ), lambda b: (b, 0)),
        pl.BlockSpec((SB * T_H * TRAJ, N_NEI), lambda b: (b, 0)),
        pl.BlockSpec((SB * T_H, TRAJ), lambda b: (b, 0)),
        pl.BlockSpec((NTOKH, D_NOISE), lambda b: (b, 0)),
        pl.BlockSpec((NTOKH, D_NOISE), lambda b: (b, 0)),
        fixed((T_H, D)),
        fixed((T_H * TRAJ, 4)),
        fixed((T_H * TRAJ, FC)),
        fixed((T_H * TRAJ, FC)),
    ] + [fixed(cmats[n].shape) for n in _CONST_NAMES] \
      + [fixed(l.shape) for l in leaves]

    out = pl.pallas_call(
        functools.partial(_block_body, treedef),
        grid=(nblk,),
        in_specs=in_specs,
        out_specs=pl.BlockSpec((ROWS, OUTW), lambda b: (b, 0)),
        out_shape=jax.ShapeDtypeStruct((B * N_NEI, OUTW), jnp.float32),
    )(xe2, xn2, xnT, xet, ze, zn, pe, msm, myl, mb, *cvals, *leaves)
    return out.reshape(B, N_NEI, INSIGHTS, T_F, TRAJ)
